# Initial kernel scaffold; baseline (speedup 1.0000x reference)
import jax
import jax.numpy as jnp
from jax.experimental import pallas as pl


def kernel(x_a, x_b, edge_index_rel0, edge_index_rel1, W0, b0, W1, b1):
    raise NotImplementedError("write your pallas kernel here")


# speedup vs baseline: 1.7990x; 1.7990x over previous
import functools

import jax
import jax.numpy as jnp
from jax import lax
from jax.experimental import pallas as pl
from jax.experimental.pallas import tpu as pltpu
from jax.experimental.pallas import tpu_sc as plsc

N_NODES = 100000
D = 128
E_EDGES = 320000
BLK = 400
WIN = 512
SC_CH = 400


def _norm_scale_kernel(starts, src_ref, x_ref, xn_ref, sw, sem):
    i = pl.program_id(0)
    e0 = starts[i]
    e1 = starts[i + 1]
    rows = i * BLK + lax.broadcasted_iota(jnp.int32, (BLK, 1), 0)
    a0 = (e0 // WIN) * WIN
    nt = (e1 - a0 + WIN - 1) // WIN

    def body(k, deg):
        off = a0 + k * WIN
        cp = pltpu.make_async_copy(src_ref.at[:, pl.ds(off, WIN)], sw, sem)
        cp.start()
        cp.wait()
        pos = off + lax.broadcasted_iota(jnp.int32, (1, WIN), 1)
        valid = (pos >= e0) & (pos < e1)
        eq = (sw[...] == rows) & valid
        return deg + jnp.sum(eq.astype(jnp.float32), axis=1, keepdims=True)

    deg = lax.fori_loop(0, nt, body, jnp.zeros((BLK, 1), jnp.float32))
    norm = jnp.where(deg > 0, lax.rsqrt(deg), 0.0)
    xn_ref[...] = x_ref[...] * norm


def _norm_scale(x, src_sorted, starts):
    grid_spec = pltpu.PrefetchScalarGridSpec(
        num_scalar_prefetch=1,
        grid=(N_NODES // BLK,),
        in_specs=[
            pl.BlockSpec(memory_space=pl.ANY),
            pl.BlockSpec((BLK, D), lambda i, s: (i, 0)),
        ],
        out_specs=pl.BlockSpec((BLK, D), lambda i, s: (i, 0)),
        scratch_shapes=[
            pltpu.VMEM((1, WIN), jnp.int32),
            pltpu.SemaphoreType.DMA,
        ],
    )
    return pl.pallas_call(
        _norm_scale_kernel,
        grid_spec=grid_spec,
        out_shape=jax.ShapeDtypeStruct((N_NODES, D), jnp.float32),
    )(starts, src_sorted.reshape(1, E_EDGES), x)


def _sc_gather(xn, idx):
    info = plsc.get_sparse_core_info()
    nw = info.num_cores * info.num_subcores
    per_w = E_EDGES // nw
    n_ch = per_w // SC_CH
    mesh = plsc.VectorSubcoreMesh(core_axis_name="c", subcore_axis_name="s")

    @functools.partial(
        pl.kernel,
        mesh=mesh,
        out_type=jax.ShapeDtypeStruct((E_EDGES, D), jnp.float32),
        scratch_types=[
            pltpu.VMEM((SC_CH,), jnp.int32),
            pltpu.VMEM((SC_CH, D), jnp.float32),
            pltpu.SemaphoreType.DMA,
        ],
    )
    def k(xn_hbm, idx_hbm, out_hbm, idx_v, rows_v, sem):
        wid = lax.axis_index("s") * info.num_cores + lax.axis_index("c")
        base = wid * per_w
        for c in range(n_ch):
            off = base + c * SC_CH
            pltpu.sync_copy(idx_hbm.at[pl.ds(off, SC_CH)], idx_v)
            pltpu.async_copy(xn_hbm.at[idx_v], rows_v, sem).wait()
            pltpu.sync_copy(rows_v, out_hbm.at[pl.ds(off, SC_CH)])

    return k(xn, idx)


def _segsum_kernel(starts, dst_ref, m_ref, w_ref, b_ref, out_ref, mw, dw,
                   sem_m, sem_d, *, row_base):
    i = pl.program_id(0)
    e0 = starts[i]
    e1 = starts[i + 1]
    rows = row_base + i * BLK + lax.broadcasted_iota(jnp.int32, (BLK, 1), 0)
    a0 = (e0 // WIN) * WIN
    nt = (e1 - a0 + WIN - 1) // WIN

    def body(k, carry):
        acc, deg = carry
        off = a0 + k * WIN
        cm = pltpu.make_async_copy(m_ref.at[pl.ds(off, WIN)], mw, sem_m)
        cd = pltpu.make_async_copy(dst_ref.at[:, pl.ds(off, WIN)], dw, sem_d)
        cm.start()
        cd.start()
        cm.wait()
        cd.wait()
        pos = off + lax.broadcasted_iota(jnp.int32, (1, WIN), 1)
        valid = (pos >= e0) & (pos < e1)
        cmp = ((dw[...] == rows) & valid).astype(jnp.float32)
        acc = acc + jnp.dot(cmp, mw[...], preferred_element_type=jnp.float32)
        deg = deg + jnp.sum(cmp, axis=1, keepdims=True)
        return acc, deg

    acc, deg = lax.fori_loop(
        0, nt, body,
        (jnp.zeros((BLK, D), jnp.float32), jnp.zeros((BLK, 1), jnp.float32)),
    )
    norm = jnp.where(deg > 0, lax.rsqrt(deg), 0.0)
    out_ref[...] = (
        jnp.dot(acc * norm, w_ref[...], preferred_element_type=jnp.float32)
        + b_ref[...]
    )


def _segsum(m, dst_sorted, starts, W, b, row_base, n_rows):
    grid_spec = pltpu.PrefetchScalarGridSpec(
        num_scalar_prefetch=1,
        grid=(n_rows // BLK,),
        in_specs=[
            pl.BlockSpec(memory_space=pl.ANY),
            pl.BlockSpec(memory_space=pl.ANY),
            pl.BlockSpec((D, D), lambda i, s: (0, 0)),
            pl.BlockSpec((1, D), lambda i, s: (0, 0)),
        ],
        out_specs=pl.BlockSpec((BLK, D), lambda i, s: (i, 0)),
        scratch_shapes=[
            pltpu.VMEM((WIN, D), jnp.float32),
            pltpu.VMEM((1, WIN), jnp.int32),
            pltpu.SemaphoreType.DMA,
            pltpu.SemaphoreType.DMA,
        ],
    )
    return pl.pallas_call(
        functools.partial(_segsum_kernel, row_base=row_base),
        grid_spec=grid_spec,
        out_shape=jax.ShapeDtypeStruct((n_rows, D), jnp.float32),
    )(starts, dst_sorted.reshape(1, E_EDGES), m, W, b.reshape(1, D))


def _relation(x_src, src, dst, W, b, keep_lo, keep_hi):
    dst_s, src_s = lax.sort_key_val(dst, src)
    src_sorted = jnp.sort(src)
    bounds_src = (jnp.arange(N_NODES // BLK + 1, dtype=jnp.int32) * BLK)
    starts_src = jnp.searchsorted(src_sorted, bounds_src).astype(jnp.int32)
    n_keep = keep_hi - keep_lo
    bounds_dst = keep_lo + jnp.arange(n_keep // BLK + 1, dtype=jnp.int32) * BLK
    starts_dst = jnp.searchsorted(dst_s, bounds_dst).astype(jnp.int32)

    xn = _norm_scale(x_src, src_sorted, starts_src)
    m = _sc_gather(xn, src_s)
    return _segsum(m, dst_s, starts_dst, W, b, keep_lo, n_keep)


def kernel(x_a, x_b, edge_index_rel0, edge_index_rel1, W0, b0, W1, b1):
    h_b = _relation(x_a, edge_index_rel0[0], edge_index_rel0[1], W0, b0,
                    50000, 100000)
    h_a = _relation(x_b, edge_index_rel1[0], edge_index_rel1[1], W1, b1,
                    0, 50000)
    return jnp.concatenate([h_a, h_b], axis=0)
